# baseline (speedup 1.0000x reference)
import jax
import jax.numpy as jnp
from jax.experimental import pallas as pl

_TOPK = 8
_BLOCK = 32


def _fused_body(x_ref, w_ref, tri_ref, out_ref, eff_ref, st_ref):
    B, C, HW = x_ref.shape
    K = _TOPK
    wv = w_ref[...]

    rows = []
    for s in range(B):
        rows.append(
            jax.lax.dot_general(wv, x_ref[s], (((1,), (0,)), ((), ())))
        )
    t = jnp.concatenate(rows, axis=0)

    te = jnp.exp(t)
    tn = te / jnp.sum(te, axis=1, keepdims=True)

    iota = jax.lax.broadcasted_iota(jnp.int32, (B, HW), 1)
    v = tn
    mask = jnp.zeros((B, HW), dtype=jnp.bool_)
    for _ in range(K):
        m = jnp.max(v, axis=1, keepdims=True)
        first = jnp.min(jnp.where(v == m, iota, HW), axis=1, keepdims=True)
        hit = iota == first
        mask = jnp.logical_or(mask, hit)
        v = jnp.where(hit, -1.0, v)

    st = jnp.where(mask, (1.0 - tn) + tn, 0.0)
    st_ref[...] = st

    mf = jnp.where(mask, 1.0, 0.0)
    rank1 = jax.lax.dot_general(
        mf, tri_ref[...], (((1,), (0,)), ((), ()))
    )

    out_all = x_ref[...] * st[:, None, :]
    out_ref[...] = out_all

    kio = jax.lax.broadcasted_iota(jnp.int32, (K, HW), 0).astype(jnp.float32)
    for s in range(B):
        oh = jnp.where(
            (rank1[s : s + 1] == kio + 1.0) & mask[s : s + 1], 1.0, 0.0
        )
        eff_ref[s] = jax.lax.dot_general(
            oh, out_all[s], (((1,), (1,)), ((), ()))
        )


def kernel(x, w):
    N, C, H, W = x.shape
    HW = H * W
    K = _TOPK
    B = _BLOCK
    xf = x.reshape(N, C, HW)
    w2 = w.reshape(1, C)
    tri = jnp.triu(jnp.ones((HW, HW), dtype=jnp.float32))

    out_flat, eff, st_flat = pl.pallas_call(
        _fused_body,
        grid=(N // B,),
        in_specs=[
            pl.BlockSpec((B, C, HW), lambda i: (i, 0, 0)),
            pl.BlockSpec((1, C), lambda i: (0, 0)),
            pl.BlockSpec((HW, HW), lambda i: (0, 0)),
        ],
        out_specs=[
            pl.BlockSpec((B, C, HW), lambda i: (i, 0, 0)),
            pl.BlockSpec((B, K, C), lambda i: (i, 0, 0)),
            pl.BlockSpec((B, HW), lambda i: (i, 0)),
        ],
        out_shape=[
            jax.ShapeDtypeStruct((N, C, HW), x.dtype),
            jax.ShapeDtypeStruct((N, K, C), x.dtype),
            jax.ShapeDtypeStruct((N, HW), x.dtype),
        ],
    )(xf, w2, tri)

    out = out_flat.reshape(N, C, H, W)
    st_mask = st_flat.reshape(N, 1, H, W)
    concat_out = jnp.concatenate([eff.reshape(N, K * C), st_flat], axis=1)
    return concat_out, st_mask, out


# speedup vs baseline: 1.0007x; 1.0007x over previous
import jax
import jax.numpy as jnp
from jax.experimental import pallas as pl
from jax.experimental.pallas import tpu as pltpu

_TOPK = 8
_BLOCK = 32


def _fused_body(x_ref, w_ref, tri_ref, out_ref, eff_ref, st_ref):
    B, C, HW = x_ref.shape
    K = _TOPK
    wv = w_ref[...]

    rows = []
    for s in range(B):
        rows.append(
            jax.lax.dot_general(wv, x_ref[s], (((1,), (0,)), ((), ())))
        )
    t = jnp.concatenate(rows, axis=0)

    te = jnp.exp(t)
    tn = te / jnp.sum(te, axis=1, keepdims=True)

    iota = jax.lax.broadcasted_iota(jnp.int32, (B, HW), 1)
    v = tn
    mask = jnp.zeros((B, HW), dtype=jnp.bool_)
    for _ in range(K):
        m = jnp.max(v, axis=1, keepdims=True)
        first = jnp.min(jnp.where(v == m, iota, HW), axis=1, keepdims=True)
        hit = iota == first
        mask = jnp.logical_or(mask, hit)
        v = jnp.where(hit, -1.0, v)

    st = jnp.where(mask, (1.0 - tn) + tn, 0.0)
    st_ref[...] = st

    mf = jnp.where(mask, 1.0, 0.0)
    rank1 = jax.lax.dot_general(
        mf, tri_ref[...], (((1,), (0,)), ((), ()))
    )

    out_all = x_ref[...] * st[:, None, :]
    out_ref[...] = out_all

    kio = jax.lax.broadcasted_iota(jnp.int32, (K, HW), 0).astype(jnp.float32)
    for s in range(B):
        oh = jnp.where(
            (rank1[s : s + 1] == kio + 1.0) & mask[s : s + 1], 1.0, 0.0
        )
        eff_ref[s] = jax.lax.dot_general(
            oh, out_all[s], (((1,), (1,)), ((), ()))
        )


def kernel(x, w):
    N, C, H, W = x.shape
    HW = H * W
    K = _TOPK
    B = _BLOCK
    xf = x.reshape(N, C, HW)
    w2 = w.reshape(1, C)
    tri = jnp.triu(jnp.ones((HW, HW), dtype=jnp.float32))

    out_flat, eff, st_flat = pl.pallas_call(
        _fused_body,
        grid=(N // B,),
        in_specs=[
            pl.BlockSpec((B, C, HW), lambda i: (i, 0, 0)),
            pl.BlockSpec((1, C), lambda i: (0, 0)),
            pl.BlockSpec((HW, HW), lambda i: (0, 0)),
        ],
        out_specs=[
            pl.BlockSpec((B, C, HW), lambda i: (i, 0, 0)),
            pl.BlockSpec((B, K, C), lambda i: (i, 0, 0)),
            pl.BlockSpec((B, HW), lambda i: (i, 0)),
        ],
        out_shape=[
            jax.ShapeDtypeStruct((N, C, HW), x.dtype),
            jax.ShapeDtypeStruct((N, K, C), x.dtype),
            jax.ShapeDtypeStruct((N, HW), x.dtype),
        ],
        compiler_params=pltpu.CompilerParams(
            dimension_semantics=("parallel",),
        ),
    )(xf, w2, tri)

    out = out_flat.reshape(N, C, H, W)
    st_mask = st_flat.reshape(N, 1, H, W)
    concat_out = jnp.concatenate([eff.reshape(N, K * C), st_flat], axis=1)
    return concat_out, st_mask, out
